# Initial kernel scaffold; baseline (speedup 1.0000x reference)
import jax
import jax.numpy as jnp
from jax.experimental import pallas as pl


def kernel(x_tensor, E):
    raise NotImplementedError("write your pallas kernel here")


# speedup vs baseline: 1.8547x; 1.8547x over previous
import functools

import jax
import jax.numpy as jnp
from jax import lax
from jax.experimental import pallas as pl
from jax.experimental.pallas import tpu as pltpu
from jax.experimental.pallas import tpu_sc as plsc


def _pick_chunk(bpw: int, d: int) -> int:
    budget = 500_000
    best = 8
    for c in range(8, bpw + 1, 8):
        if bpw % c:
            continue
        if c * 4 + c * d * 4 <= budget:
            best = c
    return best


@functools.lru_cache(maxsize=None)
def _build(n_rows: int, vocab: int, d: int):
    info = plsc.get_sparse_core_info()
    nw = info.num_cores * info.num_subcores
    nc = info.num_cores
    assert n_rows % nw == 0
    bpw = n_rows // nw
    chunk = _pick_chunk(bpw, d)
    nchunk = bpw // chunk

    mesh = plsc.VectorSubcoreMesh(core_axis_name="c", subcore_axis_name="s")

    @functools.partial(
        pl.kernel,
        out_type=jax.ShapeDtypeStruct((n_rows, d), jnp.float32),
        mesh=mesh,
        scratch_types=[
            pltpu.VMEM((chunk,), jnp.int32),
            pltpu.VMEM((chunk, d), jnp.float32),
            pltpu.SemaphoreType.DMA,
        ],
        compiler_params=pltpu.CompilerParams(use_tc_tiling_on_sc=False),
    )
    def gather_kernel(idx_hbm, table_hbm, out_hbm, idx_v, rows_v, sem):
        wid = lax.axis_index("s") * nc + lax.axis_index("c")
        base = wid * bpw

        def body(g, carry):
            off = base + g * chunk
            pltpu.sync_copy(idx_hbm.at[pl.ds(off, chunk)], idx_v)
            pltpu.async_copy(table_hbm.at[idx_v], rows_v, sem).wait()
            pltpu.sync_copy(rows_v, out_hbm.at[pl.ds(off, chunk)])
            return carry

        lax.fori_loop(0, nchunk, body, 0)

    return gather_kernel


def kernel(x_tensor, E):
    batch, hist = x_tensor.shape
    vocab, d = E.shape
    flat_idx = x_tensor.reshape(-1)
    out = _build(flat_idx.shape[0], vocab, d)(flat_idx, E)
    return out.reshape(batch, hist, d)


# speedup vs baseline: 1.8660x; 1.0061x over previous
import functools

import jax
import jax.numpy as jnp
from jax import lax
from jax.experimental import pallas as pl
from jax.experimental.pallas import tpu as pltpu
from jax.experimental.pallas import tpu_sc as plsc

_NBUF = 4


def _pick_chunk(bpw: int, d: int) -> int:
    budget = 490_000
    best = 8
    for c in range(8, bpw + 1, 8):
        if bpw % c:
            continue
        if _NBUF * (c * 4 + c * d * 4) <= budget:
            best = c
    return best


@functools.lru_cache(maxsize=None)
def _build(n_rows: int, vocab: int, d: int):
    info = plsc.get_sparse_core_info()
    nw = info.num_cores * info.num_subcores
    nc = info.num_cores
    assert n_rows % nw == 0
    bpw = n_rows // nw
    chunk = _pick_chunk(bpw, d)
    nchunk = bpw // chunk

    mesh = plsc.VectorSubcoreMesh(core_axis_name="c", subcore_axis_name="s")

    @functools.partial(
        pl.kernel,
        out_type=jax.ShapeDtypeStruct((n_rows, d), jnp.float32),
        mesh=mesh,
        scratch_types=[
            pltpu.VMEM((_NBUF, chunk), jnp.int32),
            pltpu.VMEM((_NBUF, chunk, d), jnp.float32),
            pltpu.SemaphoreType.DMA((_NBUF,)),
            pltpu.SemaphoreType.DMA((_NBUF,)),
        ],
        compiler_params=pltpu.CompilerParams(use_tc_tiling_on_sc=False),
    )
    def gather_kernel(idx_hbm, table_hbm, out_hbm, idx_v, rows_v, gsem, ssem):
        wid = lax.axis_index("s") * nc + lax.axis_index("c")
        base = wid * bpw

        def load_idx(g, b):
            pltpu.sync_copy(idx_hbm.at[pl.ds(base + g * chunk, chunk)],
                            idx_v.at[b])

        def gather_copy(b):
            return pltpu.make_async_copy(table_hbm.at[idx_v.at[b]],
                                         rows_v.at[b], gsem.at[b])

        def store_copy(g, b):
            return pltpu.make_async_copy(
                rows_v.at[b], out_hbm.at[pl.ds(base + g * chunk, chunk)],
                ssem.at[b])

        def body(g, carry):
            b = lax.rem(g, _NBUF)

            @pl.when(g >= _NBUF)
            def _():
                store_copy(g - _NBUF, b).wait()

            load_idx(g, b)
            gather_copy(b).start()

            @pl.when(g >= 2)
            def _():
                b2 = lax.rem(g - 2, _NBUF)
                gather_copy(b2).wait()
                store_copy(g - 2, b2).start()

            return carry

        lax.fori_loop(0, nchunk, body, 0)

        for g in range(nchunk - 2, nchunk):
            b = g % _NBUF
            gather_copy(b).wait()
            store_copy(g, b).start()
        for g in range(nchunk - _NBUF, nchunk):
            store_copy(g, g % _NBUF).wait()

    return gather_kernel


def kernel(x_tensor, E):
    batch, hist = x_tensor.shape
    vocab, d = E.shape
    flat_idx = x_tensor.reshape(-1)
    out = _build(flat_idx.shape[0], vocab, d)(flat_idx, E)
    return out.reshape(batch, hist, d)


# speedup vs baseline: 1.9512x; 1.0456x over previous
import functools

import jax
import jax.numpy as jnp
from jax import lax
from jax.experimental import pallas as pl
from jax.experimental.pallas import tpu as pltpu
from jax.experimental.pallas import tpu_sc as plsc

_NBUF = 4


def _pick_chunk(bpw: int, d: int) -> int:
    budget = 490_000
    best = 8
    for c in range(8, bpw + 1, 8):
        if bpw % c:
            continue
        if _NBUF * (c * 4 + c * d * 4) <= budget:
            best = c
    return best


@functools.lru_cache(maxsize=None)
def _build(n_rows: int, vocab: int, d: int):
    info = plsc.get_sparse_core_info()
    nw = info.num_cores * info.num_subcores
    nc = info.num_cores
    assert n_rows % nw == 0
    bpw = n_rows // nw
    chunk = _pick_chunk(bpw, d)
    nchunk = bpw // chunk

    mesh = plsc.VectorSubcoreMesh(core_axis_name="c", subcore_axis_name="s")

    @functools.partial(
        pl.kernel,
        out_type=jax.ShapeDtypeStruct((n_rows, d), jnp.float32),
        mesh=mesh,
        scratch_types=[
            pltpu.VMEM((_NBUF, chunk), jnp.int32),
            pltpu.VMEM((_NBUF, chunk, d), jnp.float32),
            pltpu.SemaphoreType.DMA((_NBUF,)),
            pltpu.SemaphoreType.DMA((_NBUF,)),
        ],
        compiler_params=pltpu.CompilerParams(use_tc_tiling_on_sc=False),
    )
    def gather_kernel(idx_hbm, table_hbm, out_hbm, idx_v, rows_v, gsem, ssem):
        wid = lax.axis_index("s") * nc + lax.axis_index("c")
        base = wid * bpw

        def load_idx(g, b):
            pltpu.sync_copy(idx_hbm.at[pl.ds(base + g * chunk, chunk)],
                            idx_v.at[b])

        def gather_copy(b):
            return pltpu.make_async_copy(table_hbm.at[idx_v.at[b]],
                                         rows_v.at[b], gsem.at[b])

        def store_copy(g, b):
            return pltpu.make_async_copy(
                rows_v.at[b], out_hbm.at[pl.ds(base + g * chunk, chunk)],
                ssem.at[b])

        def body(g, carry):
            b = lax.rem(g, _NBUF)

            @pl.when(g >= _NBUF)
            def _():
                store_copy(g - _NBUF, b).wait()

            load_idx(g, b)
            gather_copy(b).start()

            @pl.when(g >= 2)
            def _():
                b2 = lax.rem(g - 2, _NBUF)
                gather_copy(b2).wait()
                store_copy(g - 2, b2).start()

            return carry

        lax.fori_loop(0, nchunk, body, 0)

        for g in range(nchunk - 2, nchunk):
            b = g % _NBUF
            gather_copy(b).wait()
            store_copy(g, b).start()
        for g in range(nchunk - _NBUF, nchunk):
            store_copy(g, g % _NBUF).wait()

    return gather_kernel


def kernel(x_tensor, E):
    batch, hist = x_tensor.shape
    vocab, d = E.shape
    flat_idx = x_tensor.T.reshape(-1)
    e_dense = jax.lax.optimization_barrier(E.reshape(vocab * d // 128, 128))
    table = e_dense.reshape(vocab, d)
    out = _build(flat_idx.shape[0], vocab, d)(flat_idx, table)
    return out.reshape(hist, batch, d).transpose(1, 0, 2)


# speedup vs baseline: 2.2053x; 1.1302x over previous
import functools

import jax
import jax.numpy as jnp
from jax import lax
from jax.experimental import pallas as pl
from jax.experimental.pallas import tpu as pltpu
from jax.experimental.pallas import tpu_sc as plsc

_NBUF = 4
_LANES = 128
_CB = 2048


@functools.lru_cache(maxsize=None)
def _build_relayout(vocab: int, d: int):
    grid = -(-vocab // _CB)
    n_rows = grid * _CB

    def body(et_ref, out_ref):
        out_ref[:, 0:d] = jnp.transpose(et_ref[...])

    def call(et):
        return pl.pallas_call(
            body,
            grid=(grid,),
            in_specs=[pl.BlockSpec((d, _CB), lambda g: (0, g))],
            out_specs=pl.BlockSpec((_CB, _LANES), lambda g: (g, 0)),
            out_shape=jax.ShapeDtypeStruct((n_rows, _LANES), jnp.float32),
        )(et)

    return call


def _pick_chunk(bpw: int, d: int) -> int:
    budget = 490_000
    best = 8
    for c in range(8, bpw + 1, 8):
        if bpw % c:
            continue
        if _NBUF * (c * 4 + c * d * 4) <= budget:
            best = c
    return best


@functools.lru_cache(maxsize=None)
def _build_gather(batch: int, hist: int, n_tab: int, d: int):
    info = plsc.get_sparse_core_info()
    nw = info.num_cores * info.num_subcores
    nc = info.num_cores
    assert batch % nw == 0
    bpw = batch // nw
    chunk = _pick_chunk(bpw, d)
    cph = bpw // chunk
    nchunk = hist * cph

    mesh = plsc.VectorSubcoreMesh(core_axis_name="c", subcore_axis_name="s")

    @functools.partial(
        pl.kernel,
        out_type=jax.ShapeDtypeStruct((hist, batch, d), jnp.float32),
        mesh=mesh,
        scratch_types=[
            pltpu.VMEM((_NBUF, chunk), jnp.int32),
            pltpu.VMEM((_NBUF, chunk, d), jnp.float32),
            pltpu.SemaphoreType.DMA((_NBUF,)),
            pltpu.SemaphoreType.DMA((_NBUF,)),
        ],
        compiler_params=pltpu.CompilerParams(use_tc_tiling_on_sc=False),
    )
    def gather_kernel(idx_hbm, table_hbm, out_hbm, idx_v, rows_v, gsem, ssem):
        wid = lax.axis_index("s") * nc + lax.axis_index("c")
        b_base = wid * bpw

        def load_idx(g, b):
            h = g // cph
            off = h * batch + b_base + (g % cph) * chunk
            pltpu.sync_copy(idx_hbm.at[pl.ds(off, chunk)], idx_v.at[b])

        def gather_copy(b):
            return pltpu.make_async_copy(table_hbm.at[idx_v.at[b]],
                                         rows_v.at[b], gsem.at[b])

        def store_copy(g, b):
            h = g // cph
            b0 = b_base + (g % cph) * chunk
            return pltpu.make_async_copy(
                rows_v.at[b],
                out_hbm.at[h, pl.ds(b0, chunk)], ssem.at[b])

        def body(g, carry):
            b = lax.rem(g, _NBUF)

            @pl.when(g >= _NBUF)
            def _():
                store_copy(g - _NBUF, b).wait()

            load_idx(g, b)
            gather_copy(b).start()

            @pl.when(g >= 2)
            def _():
                b2 = lax.rem(g - 2, _NBUF)
                gather_copy(b2).wait()
                store_copy(g - 2, b2).start()

            return carry

        lax.fori_loop(0, nchunk, body, 0)

        for g in range(nchunk - 2, nchunk):
            b = g % _NBUF
            gather_copy(b).wait()
            store_copy(g, b).start()
        for g in range(nchunk - _NBUF, nchunk):
            store_copy(g, g % _NBUF).wait()

    return gather_kernel


def kernel(x_tensor, E):
    batch, hist = x_tensor.shape
    vocab, d = E.shape
    flat_idx = x_tensor.T.reshape(-1)
    staged = _build_relayout(vocab, d)(E.T)
    ratio = _LANES // d
    table = staged.reshape(staged.shape[0] * ratio, d)
    out = _build_gather(batch, hist, table.shape[0], d)(
        flat_idx * ratio, table)
    return out.transpose(1, 0, 2)
